# Initial kernel scaffold; baseline (speedup 1.0000x reference)
import jax
import jax.numpy as jnp
from jax.experimental import pallas as pl


def kernel(words_seq, table):
    raise NotImplementedError("write your pallas kernel here")


# speedup vs baseline: 2.4874x; 2.4874x over previous
import functools

import jax
import jax.numpy as jnp
from jax import lax
from jax.experimental import pallas as pl
from jax.experimental.pallas import tpu as pltpu
from jax.experimental.pallas import tpu_sc as plsc

_VOCAB = 100000
_DIM = 32
_B = 4096
_L = 50
_N = _B * _L

_NC = 2
_NS = 16
_NW = _NC * _NS
_CHUNK = 128
_ROWS_PER_W = _N // _NW
_NSTREAM = _ROWS_PER_W // _CHUNK


def _make_sc_gather():
    mesh = plsc.VectorSubcoreMesh(core_axis_name="c", subcore_axis_name="s")

    @functools.partial(
        pl.kernel,
        mesh=mesh,
        out_type=jax.ShapeDtypeStruct((_N, _DIM), jnp.float32),
        scratch_types=[
            pltpu.VMEM((_NSTREAM, _CHUNK), jnp.int32),
            pltpu.VMEM((_CHUNK, _DIM), jnp.float32),
            pltpu.SemaphoreType.DMA,
        ],
        compiler_params=pltpu.CompilerParams(use_tc_tiling_on_sc=False),
    )
    def sc_gather(idx_hbm, table_hbm, out_hbm, idx_v, rows_v, gsem):
        wid = lax.axis_index("s") * _NC + lax.axis_index("c")
        base = wid * _ROWS_PER_W
        pltpu.sync_copy(idx_hbm.at[wid], idx_v)

        def body(j, carry):
            pltpu.async_copy(table_hbm.at[idx_v.at[j]], rows_v, gsem).wait()
            pltpu.sync_copy(rows_v, out_hbm.at[pl.ds(base + j * _CHUNK, _CHUNK), :])
            return carry

        lax.fori_loop(0, _NSTREAM, body, 0)

    return sc_gather


_sc_gather = _make_sc_gather()


def kernel(words_seq, table):
    idx = words_seq.reshape(_NW, _NSTREAM, _CHUNK)
    out = _sc_gather(idx, table)
    return out.reshape(_B, _L, _DIM)


# speedup vs baseline: 2.7346x; 1.0994x over previous
import functools

import jax
import jax.numpy as jnp
from jax import lax
from jax.experimental import pallas as pl
from jax.experimental.pallas import tpu as pltpu
from jax.experimental.pallas import tpu_sc as plsc

_VOCAB = 100000
_DIM = 32
_B = 4096
_L = 50
_N = _B * _L

_NC = 2
_NS = 16
_NW = _NC * _NS
_LANES = 128
_K = 5
_CHUNK = _K * _LANES
_ROWS_PER_W = _N // _NW
_NSTREAM = _ROWS_PER_W // _CHUNK
_NBUF = 4
_DEPTH = 2


def _make_sc_gather():
    mesh = plsc.VectorSubcoreMesh(core_axis_name="c", subcore_axis_name="s")

    @functools.partial(
        pl.kernel,
        mesh=mesh,
        out_type=jax.ShapeDtypeStruct((_N, _DIM), jnp.float32),
        scratch_types=[
            pltpu.VMEM((_NSTREAM, _CHUNK), jnp.int32),
            pltpu.VMEM((_NBUF, _CHUNK, _DIM), jnp.float32),
            pltpu.SemaphoreType.DMA((_NBUF,)),
            pltpu.SemaphoreType.DMA((_NBUF,)),
        ],
        compiler_params=pltpu.CompilerParams(use_tc_tiling_on_sc=False),
    )
    def sc_gather(idx_hbm, table_hbm, out_hbm, idx_v, bufs, gsem, osem):
        wid = lax.axis_index("s") * _NC + lax.axis_index("c")
        base = wid * _ROWS_PER_W
        pltpu.sync_copy(idx_hbm.at[wid], idx_v)

        def gather(j, b):
            return pltpu.make_async_copy(
                table_hbm.at[idx_v.at[j]], bufs.at[b], gsem.at[b])

        def writeback(j, b):
            return pltpu.make_async_copy(
                bufs.at[b], out_hbm.at[pl.ds(base + j * _CHUNK, _CHUNK), :],
                osem.at[b])

        for j in range(_DEPTH):
            gather(j, j % _NBUF).start()

        def body(j, carry):
            b = lax.rem(j, _NBUF)
            gather(j, b).wait()
            writeback(j, b).start()

            @pl.when(j + _DEPTH < _NSTREAM)
            def _():
                bb = lax.rem(j + _DEPTH, _NBUF)

                @pl.when(j + _DEPTH >= _NBUF)
                def _():
                    writeback(j + _DEPTH - _NBUF, bb).wait()

                gather(j + _DEPTH, bb).start()

            return carry

        lax.fori_loop(0, _NSTREAM, body, 0)

        for j in range(_NSTREAM - _NBUF, _NSTREAM):
            writeback(j, j % _NBUF).wait()

    return sc_gather


_sc_gather = _make_sc_gather()


def kernel(words_seq, table):
    idx = words_seq.reshape(_NW, _NSTREAM, _CHUNK)
    out = _sc_gather(idx, table)
    return out.reshape(_B, _L, _DIM)


# speedup vs baseline: 4.6631x; 1.7052x over previous
import functools

import jax
import jax.numpy as jnp
from jax import lax
from jax.experimental import pallas as pl
from jax.experimental.pallas import tpu as pltpu
from jax.experimental.pallas import tpu_sc as plsc

_VOCAB = 100000
_DIM = 32
_B = 4096
_L = 50

_NC = 2
_NS = 16
_NW = _NC * _NS
_BROWS = _B // _NW
_ROWS_PER_W = _BROWS * _L
_GROUP = 16
_CHUNK = _GROUP * _L
_NSTREAM = _BROWS // _GROUP
_NBUF = 4
_DEPTH = 2


def _make_sc_gather():
    mesh = plsc.VectorSubcoreMesh(core_axis_name="c", subcore_axis_name="s")

    @functools.partial(
        pl.kernel,
        mesh=mesh,
        out_type=jax.ShapeDtypeStruct((_B, _L, _DIM), jnp.float32),
        scratch_types=[
            pltpu.VMEM((_BROWS, _L), jnp.int32),
            pltpu.VMEM((_ROWS_PER_W,), jnp.int32),
            pltpu.VMEM((_NBUF, _CHUNK, _DIM), jnp.float32),
            pltpu.SemaphoreType.DMA((_NBUF,)),
            pltpu.SemaphoreType.DMA((_NBUF,)),
        ],
        compiler_params=pltpu.CompilerParams(
            use_tc_tiling_on_sc=False, needs_layout_passes=False),
    )
    def sc_gather(idx_hbm, table_hbm, out_hbm, idx_v, flat_v, bufs, gsem, osem):
        wid = lax.axis_index("s") * _NC + lax.axis_index("c")
        row0 = wid * _BROWS
        pltpu.sync_copy(idx_hbm.at[pl.ds(row0, _BROWS), :], idx_v)

        lane = lax.iota(jnp.int32, 16)

        def flatten(i, carry):
            for c in (0, 16, 32, 34):
                v = idx_v[i, pl.ds(c, 16)]
                plsc.store_scatter(flat_v, [i * _L + c + lane], v)
            return carry

        lax.fori_loop(0, _BROWS, flatten, 0)

        def gather(j, b):
            return pltpu.make_async_copy(
                table_hbm.at[flat_v.at[pl.ds(j * _CHUNK, _CHUNK)]],
                bufs.at[b], gsem.at[b])

        def writeback_copies(j, b):
            return [
                pltpu.make_async_copy(
                    bufs.at[b, pl.ds(g * _L, _L), :],
                    out_hbm.at[row0 + j * _GROUP + g],
                    osem.at[b])
                for g in range(_GROUP)
            ]

        def writeback_start(j, b):
            for c in writeback_copies(j, b):
                c.start()

        def writeback_wait(j, b):
            for c in writeback_copies(j, b):
                c.wait()

        for j in range(_DEPTH):
            gather(j, j % _NBUF).start()

        def body(j, carry):
            b = lax.rem(j, _NBUF)
            gather(j, b).wait()
            writeback_start(j, b)

            @pl.when(j + _DEPTH < _NSTREAM)
            def _():
                bb = lax.rem(j + _DEPTH, _NBUF)

                @pl.when(j + _DEPTH >= _NBUF)
                def _():
                    writeback_wait(j + _DEPTH - _NBUF, bb)

                gather(j + _DEPTH, bb).start()

            return carry

        lax.fori_loop(0, _NSTREAM, body, 0)

        for j in range(_NSTREAM - _NBUF, _NSTREAM):
            writeback_wait(j, j % _NBUF)

    return sc_gather


_sc_gather = _make_sc_gather()


def kernel(words_seq, table):
    return _sc_gather(words_seq, table)


# speedup vs baseline: 4.6956x; 1.0070x over previous
import functools

import jax
import jax.numpy as jnp
from jax import lax
from jax.experimental import pallas as pl
from jax.experimental.pallas import tpu as pltpu
from jax.experimental.pallas import tpu_sc as plsc

_VOCAB = 100000
_DIM = 32
_B = 4096
_L = 50

_NC = 2
_NS = 16
_NW = _NC * _NS
_BROWS = _B // _NW
_ROWS_PER_W = _BROWS * _L
_GROUP = 8
_CHUNK = _GROUP * _L
_NSTREAM = _BROWS // _GROUP
_NBUF = 8
_DEPTH = 3


def _make_sc_gather():
    mesh = plsc.VectorSubcoreMesh(core_axis_name="c", subcore_axis_name="s")

    @functools.partial(
        pl.kernel,
        mesh=mesh,
        out_type=jax.ShapeDtypeStruct((_B, _L, _DIM), jnp.float32),
        scratch_types=[
            pltpu.VMEM((_BROWS, _L), jnp.int32),
            pltpu.VMEM((_ROWS_PER_W,), jnp.int32),
            pltpu.VMEM((_NBUF, _CHUNK, _DIM), jnp.float32),
            pltpu.SemaphoreType.DMA((_NBUF,)),
            pltpu.SemaphoreType.DMA((_NBUF,)),
        ],
        compiler_params=pltpu.CompilerParams(
            use_tc_tiling_on_sc=False, needs_layout_passes=False),
    )
    def sc_gather(idx_hbm, table_hbm, out_hbm, idx_v, flat_v, bufs, gsem, osem):
        wid = lax.axis_index("s") * _NC + lax.axis_index("c")
        row0 = wid * _BROWS
        pltpu.sync_copy(idx_hbm.at[pl.ds(row0, _BROWS), :], idx_v)

        lane = lax.iota(jnp.int32, 16)

        def flatten(i, carry):
            for c in (0, 16, 32, 34):
                v = idx_v[i, pl.ds(c, 16)]
                plsc.store_scatter(flat_v, [i * _L + c + lane], v)
            return carry

        lax.fori_loop(0, _BROWS, flatten, 0)

        def gather(j, b):
            return pltpu.make_async_copy(
                table_hbm.at[flat_v.at[pl.ds(j * _CHUNK, _CHUNK)]],
                bufs.at[b], gsem.at[b])

        def writeback_copies(j, b):
            return [
                pltpu.make_async_copy(
                    bufs.at[b, pl.ds(g * _L, _L), :],
                    out_hbm.at[row0 + j * _GROUP + g],
                    osem.at[b])
                for g in range(_GROUP)
            ]

        def writeback_start(j, b):
            for c in writeback_copies(j, b):
                c.start()

        def writeback_wait(j, b):
            for c in writeback_copies(j, b):
                c.wait()

        for j in range(_DEPTH):
            gather(j, j % _NBUF).start()

        def body(j, carry):
            b = lax.rem(j, _NBUF)
            gather(j, b).wait()
            writeback_start(j, b)

            @pl.when(j + _DEPTH < _NSTREAM)
            def _():
                bb = lax.rem(j + _DEPTH, _NBUF)

                @pl.when(j + _DEPTH >= _NBUF)
                def _():
                    writeback_wait(j + _DEPTH - _NBUF, bb)

                gather(j + _DEPTH, bb).start()

            return carry

        lax.fori_loop(0, _NSTREAM, body, 0)

        for j in range(_NSTREAM - _NBUF, _NSTREAM):
            writeback_wait(j, j % _NBUF)

    return sc_gather


_sc_gather = _make_sc_gather()


def kernel(words_seq, table):
    return _sc_gather(words_seq, table)


# speedup vs baseline: 5.7517x; 1.2249x over previous
import functools

import jax
import jax.numpy as jnp
from jax import lax
from jax.experimental import pallas as pl
from jax.experimental.pallas import tpu as pltpu
from jax.experimental.pallas import tpu_sc as plsc

_VOCAB = 100000
_DIM = 32
_B = 4096
_L = 50

_NC = 2
_NS = 16
_NW = _NC * _NS
_BCOLS = _B // _NW
_GROUP = 16
_CHUNK = _GROUP * _L
_NSTREAM = _BCOLS // _GROUP
_NBUF = 2


def _make_sc_gather():
    mesh = plsc.VectorSubcoreMesh(core_axis_name="c", subcore_axis_name="s")

    @functools.partial(
        pl.kernel,
        mesh=mesh,
        out_type=jax.ShapeDtypeStruct((_L, _DIM, _B), jnp.float32),
        scratch_types=[
            pltpu.VMEM((_L, _BCOLS), jnp.int32),
            pltpu.VMEM((_NSTREAM * _CHUNK,), jnp.int32),
            pltpu.VMEM((_NBUF, _CHUNK, _DIM), jnp.float32),
            pltpu.VMEM((_NBUF, _L, _DIM, _GROUP), jnp.float32),
            pltpu.SemaphoreType.DMA((_NBUF,)),
            pltpu.SemaphoreType.DMA((_NBUF,)),
        ],
        compiler_params=pltpu.CompilerParams(
            use_tc_tiling_on_sc=False, needs_layout_passes=False),
    )
    def sc_gather(idx_hbm, table_hbm, out_hbm, idx_v, flat_v, bufs, tbufs,
                  gsem, osem):
        wid = lax.axis_index("s") * _NC + lax.axis_index("c")
        col0 = wid * _BCOLS
        pltpu.sync_copy(idx_hbm.at[:, pl.ds(col0, _BCOLS)], idx_v)

        def build(i, carry):
            j = i // _L
            l = i - j * _L
            v = idx_v[l, pl.ds(j * _GROUP, _GROUP)]
            flat_v[pl.ds(j * _CHUNK + l * _GROUP, _GROUP)] = v
            return carry

        lax.fori_loop(0, _NSTREAM * _L, build, 0)

        def gather(j, b):
            return pltpu.make_async_copy(
                table_hbm.at[flat_v.at[pl.ds(j * _CHUNK, _CHUNK)]],
                bufs.at[b], gsem.at[b])

        def writeback(j, b):
            return pltpu.make_async_copy(
                tbufs.at[b],
                out_hbm.at[:, :, pl.ds(col0 + j * _GROUP, _GROUP)],
                osem.at[b])

        lane = lax.iota(jnp.int32, 16)

        def transpose(b):
            def trow(l, carry):
                lv = jnp.full((16,), 0, jnp.int32) + l
                for g in range(_GROUP):
                    gv = jnp.full((16,), g, jnp.int32)
                    for d0 in (0, 16):
                        v = bufs[b, l * _GROUP + g, pl.ds(d0, 16)]
                        plsc.store_scatter(
                            tbufs.at[b], [lv, d0 + lane, gv], v)
                return carry

            lax.fori_loop(0, _L, trow, 0)

        for j in range(_NBUF):
            gather(j, j).start()

        def body(j, carry):
            b = lax.rem(j, _NBUF)
            gather(j, b).wait()

            @pl.when(j >= _NBUF)
            def _():
                writeback(j - _NBUF, b).wait()

            transpose(b)
            writeback(j, b).start()

            @pl.when(j + _NBUF < _NSTREAM)
            def _():
                gather(j + _NBUF, b).start()

            return carry

        lax.fori_loop(0, _NSTREAM, body, 0)

        for j in range(_NSTREAM - _NBUF, _NSTREAM):
            writeback(j, j % _NBUF).wait()

    return sc_gather


_sc_gather = _make_sc_gather()


def kernel(words_seq, table):
    out_t = _sc_gather(words_seq.T, table)
    return jnp.transpose(out_t, (2, 0, 1))
